# Initial kernel scaffold; baseline (speedup 1.0000x reference)
import jax
import jax.numpy as jnp
from jax.experimental import pallas as pl


def kernel(x, edge_index, edge_attr, batch, W1, b1, W2, b2, LW1, Lb1, LW2, Lb2):
    raise NotImplementedError("write your pallas kernel here")


# speedup vs baseline: 3.9695x; 3.9695x over previous
import functools

import jax
import jax.numpy as jnp
from jax import lax
from jax.experimental import pallas as pl
from jax.experimental.pallas import tpu as pltpu
from jax.experimental.pallas import tpu_sc as plsc

N = 50000
E = 800000
G = 16

NP = 50176
EP = 819200
ER = EP // 128
TILES = 32
ROWS_PER_TILE = ER // TILES
BLK = 8
NBLK = ROWS_PER_TILE // BLK
STRIPE = NP // 16

_mesh = plsc.VectorSubcoreMesh(core_axis_name="c", subcore_axis_name="s")
_sc_params = pltpu.CompilerParams(use_tc_tiling_on_sc=False)
f32 = jnp.float32
i32 = jnp.int32
_HI = lax.Precision.HIGHEST


def _sigmoid(x):
    ax = jnp.abs(x)
    e = jnp.exp(-ax)
    pos = 1.0 / (1.0 + e)
    return jnp.where(x >= 0, pos, 1.0 - pos)


def _zero_vec(ref, nwords):
    def body(j, _):
        ref[pl.ds(16 * j, 16)] = jnp.zeros((16,), f32)
        return 0
    lax.fori_loop(0, nwords // 16, body, 0)


_GDN = lax.GatherDimensionNumbers(
    offset_dims=(), collapsed_slice_dims=(0,), start_index_map=(0,))


def _splat(vec16, lane):
    idx = jnp.full((16, 1), lane, i32)
    return lax.gather(vec16, idx, _GDN, slice_sizes=(1,),
                      mode=lax.GatherScatterMode.PROMISE_IN_BOUNDS)


@functools.partial(
    pl.kernel,
    out_type=jax.ShapeDtypeStruct((2 * NP,), f32),
    mesh=_mesh,
    compiler_params=_sc_params,
    scratch_types=[
        pltpu.VMEM_SHARED((NP,), f32),
        pltpu.VMEM((BLK, 128), i32),
        pltpu.VMEM((BLK, 128), f32),
        pltpu.VMEM((STRIPE,), f32),
    ],
)
def _sc_deg(dst_h, w_h, degp_h, acc, idst, wv, zv):
    cid = lax.axis_index("c")
    sid = lax.axis_index("s")
    _zero_vec(zv, STRIPE)
    pltpu.sync_copy(zv, acc.at[pl.ds(STRIPE * sid, STRIPE)])
    plsc.subcore_barrier()
    row0 = (cid * 16 + sid) * ROWS_PER_TILE

    def block(b, _):
        r = row0 + b * BLK
        pltpu.sync_copy(dst_h.at[pl.ds(r, BLK)], idst)
        pltpu.sync_copy(w_h.at[pl.ds(r, BLK)], wv)

        def chunk(c, _):
            pltpu.sync_copy(wv.at[c], acc.at[idst.at[c]], add=True)
            return 0

        lax.fori_loop(0, BLK, chunk, 0)
        return 0

    lax.fori_loop(0, NBLK, block, 0)
    plsc.subcore_barrier()
    pltpu.sync_copy(acc.at[pl.ds(STRIPE * sid, STRIPE)], zv)
    pltpu.sync_copy(zv, degp_h.at[pl.ds(NP * cid + STRIPE * sid, STRIPE)])


@functools.partial(
    pl.kernel,
    out_type=(
        jax.ShapeDtypeStruct((EP,), f32),
        jax.ShapeDtypeStruct((2 * NP,), f32),
        jax.ShapeDtypeStruct((2 * NP,), f32),
        jax.ShapeDtypeStruct((2 * NP,), f32),
    ),
    mesh=_mesh,
    compiler_params=_sc_params,
    scratch_types=[
        pltpu.VMEM_SHARED((NP,), f32),
        pltpu.VMEM_SHARED((NP,), f32),
        pltpu.VMEM_SHARED((NP,), f32),
        pltpu.VMEM((BLK, 128), i32),
        pltpu.VMEM((BLK, 128), i32),
        pltpu.VMEM((BLK, 128), f32),
        pltpu.VMEM((BLK, 128), f32),
        pltpu.VMEM((BLK, 128), f32),
        pltpu.VMEM((BLK, 128), f32),
        pltpu.VMEM((BLK, 128), f32),
        pltpu.VMEM((STRIPE,), f32),
    ],
)
def _sc_l1(src_h, dst_h, w_h, dis_h, x0_h, x1_h, x2_h,
           v_h, a0_h, a1_h, a2_h,
           acc0, acc1, acc2, isrc, idst, wv, dv, vv, gx, msg, zv):
    cid = lax.axis_index("c")
    sid = lax.axis_index("s")
    _zero_vec(zv, STRIPE)
    for acc in (acc0, acc1, acc2):
        pltpu.sync_copy(zv, acc.at[pl.ds(STRIPE * sid, STRIPE)])
    plsc.subcore_barrier()
    row0 = (cid * 16 + sid) * ROWS_PER_TILE

    def ew_mul(dst_ref, a_ref, b_ref):
        def body(j, _):
            rr = j >> 3
            sl = pl.ds(16 * (j & 7), 16)
            dst_ref[rr, sl] = a_ref[rr, sl] * b_ref[rr, sl]
            return 0
        lax.fori_loop(0, BLK * 8, body, 0)

    def block(b, _):
        r = row0 + b * BLK
        pltpu.sync_copy(src_h.at[pl.ds(r, BLK)], isrc)
        pltpu.sync_copy(dst_h.at[pl.ds(r, BLK)], idst)
        pltpu.sync_copy(w_h.at[pl.ds(r, BLK)], wv)

        def dchunk(c, _):
            pltpu.sync_copy(dis_h.at[isrc.at[c]], dv.at[c])
            return 0

        lax.fori_loop(0, BLK, dchunk, 0)
        ew_mul(vv, wv, dv)

        def vchunk(c, _):
            off = pl.multiple_of((r + c) * 128, 128)
            pltpu.sync_copy(vv.at[c], v_h.at[pl.ds(off, 128)])
            return 0

        lax.fori_loop(0, BLK, vchunk, 0)
        for x_h, acc in ((x0_h, acc0), (x1_h, acc1), (x2_h, acc2)):
            def xchunk(c, _):
                pltpu.sync_copy(x_h.at[isrc.at[c]], gx.at[c])
                return 0

            lax.fori_loop(0, BLK, xchunk, 0)
            ew_mul(msg, gx, vv)

            def schunk(c, _):
                pltpu.sync_copy(msg.at[c], acc.at[idst.at[c]], add=True)
                return 0

            lax.fori_loop(0, BLK, schunk, 0)
        return 0

    lax.fori_loop(0, NBLK, block, 0)
    plsc.subcore_barrier()
    for acc, out in ((acc0, a0_h), (acc1, a1_h), (acc2, a2_h)):
        pltpu.sync_copy(acc.at[pl.ds(STRIPE * sid, STRIPE)], zv)
        pltpu.sync_copy(zv, out.at[pl.ds(NP * cid + STRIPE * sid, STRIPE)])


FCH = 112
ER2 = EP // 64
SUPER = ROWS_PER_TILE // 8


@functools.partial(
    pl.kernel,
    out_type=jax.ShapeDtypeStruct((8 * NP, 32), f32),
    mesh=_mesh,
    compiler_params=_sc_params,
    scratch_types=[
        pltpu.VMEM_SHARED((NP, 32), f32),
        pltpu.VMEM((16, 64), i32),
        pltpu.VMEM((16, 64), i32),
        pltpu.VMEM((1024,), f32),
        pltpu.VMEM((64, 128), f32),
        pltpu.VMEM((64, 32), f32),
        pltpu.VMEM((FCH, 32), f32),
    ],
)
def _sc_l2(src_h, dst_h, v_h, h_h, a2p_h, acc, isrc, idst, vv, g, msg, zf):
    cid = lax.axis_index("c")
    sid = lax.axis_index("s")
    row0 = (cid * 16 + sid) * (ROWS_PER_TILE * 2)

    for p in range(4):
        def zbody(j, _):
            zf[j >> 1, pl.ds(16 * (j & 1), 16)] = jnp.zeros((16,), f32)
            return 0
        lax.fori_loop(0, FCH * 2, zbody, 0)
        def zq(q, _):
            off = pl.multiple_of(STRIPE * sid + FCH * q, 8)
            pltpu.sync_copy(zf, acc.at[pl.ds(off, FCH)])
            return 0

        lax.fori_loop(0, 28, zq, 0)
        plsc.subcore_barrier()

        def superblock(b, _):
            r = row0 + b * 16
            pltpu.sync_copy(src_h.at[pl.ds(r, 16)], isrc)
            pltpu.sync_copy(dst_h.at[pl.ds(r, 16)], idst)
            voff = pl.multiple_of(r * 64, 1024)
            pltpu.sync_copy(v_h.at[pl.ds(voff, 1024)], vv)

            def chunk(c, _):
                pltpu.sync_copy(h_h.at[isrc.at[c]], g)

                def scale(q2, _):
                    so = pl.multiple_of(64 * c + 16 * q2, 16)
                    vseg = vv[pl.ds(so, 16)]
                    for l in range(16):
                        s16 = _splat(vseg, l)
                        i = 16 * q2 + l
                        for h in range(2):
                            msg[i, pl.ds(16 * h, 16)] = (
                                g[i, pl.ds(32 * p + 16 * h, 16)] * s16)
                    return 0

                lax.fori_loop(0, 4, scale, 0)
                pltpu.sync_copy(msg, acc.at[idst.at[c]], add=True)
                return 0

            lax.fori_loop(0, 16, chunk, 0)
            return 0

        lax.fori_loop(0, SUPER, superblock, 0)
        plsc.subcore_barrier()
        obase = (cid * 4 + p) * NP

        def fq(q, _):
            off = pl.multiple_of(STRIPE * sid + FCH * q, 8)
            pltpu.sync_copy(acc.at[pl.ds(off, FCH)], zf)
            pltpu.sync_copy(zf, a2p_h.at[pl.ds(obase + off, FCH)])
            return 0

        lax.fori_loop(0, 28, fq, 0)
        plsc.subcore_barrier()


def _tc_dis_body(degp_ref, dis_ref, dis2_ref):
    x = degp_ref[0] + degp_ref[1] + 1.0
    d = lax.rsqrt(x)
    d = d * (1.5 - 0.5 * x * d * d)
    dis_ref[...] = d
    dis2_ref[...] = d * d


def _tc_dis(degp):
    degp3 = degp.reshape(2, 392, 128)
    dis, dis2 = pl.pallas_call(
        _tc_dis_body,
        grid=(7,),
        in_specs=[pl.BlockSpec((2, 56, 128), lambda i: (0, i, 0))],
        out_specs=[pl.BlockSpec((56, 128), lambda i: (i, 0)),
                   pl.BlockSpec((56, 128), lambda i: (i, 0))],
        out_shape=[jax.ShapeDtypeStruct((392, 128), f32),
                   jax.ShapeDtypeStruct((392, 128), f32)],
    )(degp3)
    return dis.reshape(NP), dis2.reshape(NP)


def _tc_h1_body(a0_ref, a1_ref, a2_ref, dis_ref, dis2_ref,
                x0_ref, x1_ref, x2_ref, w1_ref, b1_ref, o_ref):
    dis = dis_ref[...]
    dis2 = dis2_ref[...]
    h = jnp.broadcast_to(b1_ref[...], (1024, 128))
    for a_ref, x_ref, fidx in ((a0_ref, x0_ref, 0), (a1_ref, x1_ref, 1),
                               (a2_ref, x2_ref, 2)):
        agg = dis * (a_ref[0] + a_ref[1]) + dis2 * x_ref[...]
        h = h + agg * w1_ref[fidx:fidx + 1, :]
    o_ref[...] = h * _sigmoid(h)


def _tc_h1(a0, a1, a2, dis, dis2, x0, x1, x2, w1p, b1p):
    col = pl.BlockSpec((1024, 1), lambda i: (i, 0))
    par = pl.BlockSpec((2, 1024, 1), lambda i: (0, i, 0))
    return pl.pallas_call(
        _tc_h1_body,
        grid=(49,),
        in_specs=[par, par, par, col, col, col, col, col,
                  pl.BlockSpec((3, 128), lambda i: (0, 0)),
                  pl.BlockSpec((1, 128), lambda i: (0, 0))],
        out_specs=pl.BlockSpec((1024, 128), lambda i: (i, 0)),
        out_shape=jax.ShapeDtypeStruct((NP, 128), f32),
    )(a0.reshape(2, NP, 1), a1.reshape(2, NP, 1), a2.reshape(2, NP, 1),
      dis.reshape(NP, 1), dis2.reshape(NP, 1),
      x0.reshape(NP, 1), x1.reshape(NP, 1), x2.reshape(NP, 1), w1p, b1p)


def _tc_final_body(a2p_ref, h_ref,
                   dis_ref, dis2_ref, batch_ref, w2_ref, b2_ref,
                   lw1_ref, lb1_ref, lw2_ref, lb2_ref,
                   out_ref, sums, cnt):
    pid = pl.program_id(0)

    @pl.when(pid == 0)
    def _():
        sums[...] = jnp.zeros((16, 256), f32)
        cnt[...] = jnp.zeros((16, 128), f32)

    dis = dis_ref[...]
    dis2 = dis2_ref[...]
    acc = jnp.broadcast_to(b2_ref[...], (1024, 256))
    for p in range(4):
        aggp = (dis * (a2p_ref[0, p] + a2p_ref[1, p])
                + dis2 * h_ref[:, 32 * p:32 * p + 32])
        acc = acc + jnp.dot(aggp, w2_ref[32 * p:32 * p + 32, :],
                            precision=_HI, preferred_element_type=f32)
    h2 = acc * _sigmoid(acc)
    bb = batch_ref[...]
    io = lax.broadcasted_iota(i32, (1024, 16), 1)
    oh = jnp.where(bb == io, 1.0, 0.0).astype(f32)
    dn = (((0,), (0,)), ((), ()))
    sums[...] += lax.dot_general(oh, h2, dn, precision=_HI,
                                 preferred_element_type=f32)
    ones = jnp.ones((1024, 128), f32)
    cnt[...] += lax.dot_general(oh, ones, dn, precision=_HI,
                                preferred_element_type=f32)

    @pl.when(pid == 48)
    def _():
        pooled = sums[...] / jnp.maximum(cnt[...][:, 0:1], 1.0)
        ph = jnp.dot(pooled, lw1_ref[...], precision=_HI,
                     preferred_element_type=f32)
        ph = ph + lb1_ref[...]
        ph = ph * _sigmoid(ph)
        res = jnp.dot(ph, lw2_ref[...], precision=_HI,
                      preferred_element_type=f32)
        out_ref[...] = res + lb2_ref[...]


def _tc_final(a2p, h1p, dis, dis2, batchp, w2p, b2p, lw1p, lb1p, lw2p, lb2p):
    col = pl.BlockSpec((1024, 1), lambda i: (i, 0))
    whole = lambda *shape: pl.BlockSpec(shape, lambda i: tuple(0 for _ in shape))
    return pl.pallas_call(
        _tc_final_body,
        grid=(49,),
        in_specs=[pl.BlockSpec((2, 4, 1024, 32), lambda i: (0, 0, i, 0)),
                  pl.BlockSpec((1024, 128), lambda i: (i, 0)), col, col, col,
                  whole(128, 256), whole(1, 256),
                  whole(256, 128), whole(1, 128),
                  whole(128, 128), whole(1, 128)],
        out_specs=pl.BlockSpec((16, 128), lambda i: (0, 0)),
        out_shape=jax.ShapeDtypeStruct((16, 128), f32),
        scratch_shapes=[pltpu.VMEM((16, 256), f32),
                        pltpu.VMEM((16, 128), f32)],
    )(a2p, h1p, dis.reshape(NP, 1), dis2.reshape(NP, 1), batchp,
      w2p, b2p, lw1p, lb1p, lw2p, lb2p)


def kernel(x, edge_index, edge_attr, batch, W1, b1, W2, b2,
           LW1, Lb1, LW2, Lb2):
    src = edge_index[0].astype(i32)
    dst = edge_index[1].astype(i32)
    w = edge_attr.astype(f32)
    pad = EP - E
    src2 = jnp.concatenate([src, jnp.zeros((pad,), i32)]).reshape(ER, 128)
    dst2 = jnp.concatenate([dst, jnp.zeros((pad,), i32)]).reshape(ER, 128)
    w2e = jnp.concatenate([w, jnp.zeros((pad,), f32)]).reshape(ER, 128)

    degp = _sc_deg(dst2, w2e)
    dis, dis2 = _tc_dis(degp)

    xp = jnp.pad(x, ((0, NP - N), (0, 0)))
    x0, x1, x2 = xp[:, 0], xp[:, 1], xp[:, 2]
    v2, a0, a1, a2 = _sc_l1(src2, dst2, w2e, dis, x0, x1, x2)

    w1p = jnp.pad(W1, ((0, 0), (0, 28)))
    b1p = jnp.pad(b1, (0, 28)).reshape(1, 128)
    h1p = _tc_h1(a0, a1, a2, dis, dis2, x0, x1, x2, w1p, b1p)

    a2p = _sc_l2(src2.reshape(EP // 64, 64), dst2.reshape(EP // 64, 64),
                 v2, h1p).reshape(2, 4, NP, 32)

    batchp = jnp.concatenate(
        [batch.astype(i32), jnp.full((NP - N,), G, i32)]).reshape(NP, 1)
    w2p = jnp.pad(W2, ((0, 28), (0, 56)))
    b2p = jnp.pad(b2, (0, 56)).reshape(1, 256)
    lw1p = jnp.pad(LW1, ((0, 56), (0, 28)))
    lb1p = jnp.pad(Lb1, (0, 28)).reshape(1, 128)
    lw2p = jnp.pad(LW2, ((0, 28), (0, 127)))
    lb2p = jnp.pad(Lb2, (0, 127)).reshape(1, 128)

    outf = _tc_final(a2p, h1p, dis, dis2, batchp,
                     w2p, b2p, lw1p, lb1p, lw2p, lb2p)
    return outf[:, 0]


# speedup vs baseline: 5.4769x; 1.3798x over previous
import functools

import jax
import jax.numpy as jnp
from jax import lax
from jax.experimental import pallas as pl
from jax.experimental.pallas import tpu as pltpu
from jax.experimental.pallas import tpu_sc as plsc

N = 50000
E = 800000
G = 16

NP = 50176
EP = 819200
ER = EP // 128
TILES = 32
ROWS_PER_TILE = ER // TILES
BLK = 8
NBLK = ROWS_PER_TILE // BLK
STRIPE = NP // 16

_mesh = plsc.VectorSubcoreMesh(core_axis_name="c", subcore_axis_name="s")
_sc_params = pltpu.CompilerParams(use_tc_tiling_on_sc=False)
f32 = jnp.float32
i32 = jnp.int32
_HI = lax.Precision.HIGHEST


def _sigmoid(x):
    ax = jnp.abs(x)
    e = jnp.exp(-ax)
    pos = 1.0 / (1.0 + e)
    return jnp.where(x >= 0, pos, 1.0 - pos)


def _zero_vec(ref, nwords):
    def body(j, _):
        ref[pl.ds(16 * j, 16)] = jnp.zeros((16,), f32)
        return 0
    lax.fori_loop(0, nwords // 16, body, 0)


_GDN = lax.GatherDimensionNumbers(
    offset_dims=(), collapsed_slice_dims=(0,), start_index_map=(0,))


def _splat(vec16, lane):
    idx = jnp.full((16, 1), lane, i32)
    return lax.gather(vec16, idx, _GDN, slice_sizes=(1,),
                      mode=lax.GatherScatterMode.PROMISE_IN_BOUNDS)


@functools.partial(
    pl.kernel,
    out_type=jax.ShapeDtypeStruct((2 * NP,), f32),
    mesh=_mesh,
    compiler_params=_sc_params,
    scratch_types=[
        pltpu.VMEM_SHARED((NP,), f32),
        pltpu.VMEM((BLK, 128), i32),
        pltpu.VMEM((BLK, 128), f32),
        pltpu.VMEM((STRIPE,), f32),
        pltpu.SemaphoreType.DMA,
    ],
)
def _sc_deg(dst_h, w_h, degp_h, acc, idst, wv, zv, sa):
    cid = lax.axis_index("c")
    sid = lax.axis_index("s")
    _zero_vec(zv, STRIPE)
    pltpu.sync_copy(zv, acc.at[pl.ds(STRIPE * sid, STRIPE)])
    plsc.subcore_barrier()
    row0 = (cid * 16 + sid) * ROWS_PER_TILE

    def block(b, _):
        r = row0 + b * BLK
        pltpu.sync_copy(dst_h.at[pl.ds(r, BLK)], idst)
        pltpu.sync_copy(w_h.at[pl.ds(r, BLK)], wv)

        def chunk(c, _):
            pltpu.async_copy(wv.at[c], acc.at[idst.at[c]], sa, add=True)
            return 0

        lax.fori_loop(0, BLK, chunk, 0)

        def drain(c, _):
            pltpu.make_async_copy(wv.at[0], acc.at[pl.ds(0, 128)], sa).wait()
            return 0

        lax.fori_loop(0, BLK, drain, 0)
        return 0

    lax.fori_loop(0, NBLK, block, 0)
    plsc.subcore_barrier()
    pltpu.sync_copy(acc.at[pl.ds(STRIPE * sid, STRIPE)], zv)
    pltpu.sync_copy(zv, degp_h.at[pl.ds(NP * cid + STRIPE * sid, STRIPE)])


@functools.partial(
    pl.kernel,
    out_type=(
        jax.ShapeDtypeStruct((ER, 128), f32),
        jax.ShapeDtypeStruct((2 * NP,), f32),
        jax.ShapeDtypeStruct((2 * NP,), f32),
        jax.ShapeDtypeStruct((2 * NP,), f32),
    ),
    mesh=_mesh,
    compiler_params=_sc_params,
    scratch_types=[
        pltpu.VMEM_SHARED((NP,), f32),
        pltpu.VMEM_SHARED((NP,), f32),
        pltpu.VMEM_SHARED((NP,), f32),
        pltpu.VMEM((BLK, 128), i32),
        pltpu.VMEM((BLK, 128), i32),
        pltpu.VMEM((BLK, 128), f32),
        pltpu.VMEM((BLK, 128), f32),
        pltpu.VMEM((BLK, 128), f32),
        pltpu.VMEM((BLK, 128), f32),
        pltpu.VMEM((BLK, 128), f32),
        pltpu.VMEM((BLK, 128), f32),
        pltpu.VMEM((BLK, 128), f32),
        pltpu.VMEM((BLK, 128), f32),
        pltpu.VMEM((BLK, 128), f32),
        pltpu.VMEM((STRIPE,), f32),
        pltpu.SemaphoreType.DMA,
        pltpu.SemaphoreType.DMA,
    ],
)
def _sc_l1(src_h, dst_h, w_h, dis_h, x0_h, x1_h, x2_h,
           v_h, a0_h, a1_h, a2_h,
           acc0, acc1, acc2, isrc, idst, wv, dv, vv,
           gx0, gx1, gx2, ms0, ms1, ms2, zv, sd, sa):
    cid = lax.axis_index("c")
    sid = lax.axis_index("s")
    _zero_vec(zv, STRIPE)
    for acc in (acc0, acc1, acc2):
        pltpu.sync_copy(zv, acc.at[pl.ds(STRIPE * sid, STRIPE)])
    plsc.subcore_barrier()
    row0 = (cid * 16 + sid) * ROWS_PER_TILE

    def ew_mul(dst_ref, a_ref, b_ref):
        def body(j, _):
            rr = j >> 3
            sl = pl.ds(16 * (j & 7), 16)
            dst_ref[rr, sl] = a_ref[rr, sl] * b_ref[rr, sl]
            return 0
        lax.fori_loop(0, BLK * 8, body, 0)

    def block(b, _):
        r = row0 + b * BLK
        pltpu.sync_copy(src_h.at[pl.ds(r, BLK)], isrc)
        pltpu.sync_copy(dst_h.at[pl.ds(r, BLK)], idst)
        pltpu.sync_copy(w_h.at[pl.ds(r, BLK)], wv)

        def fire_g(c, _):
            pltpu.async_copy(dis_h.at[isrc.at[c]], dv.at[c], sd)
            pltpu.async_copy(x0_h.at[isrc.at[c]], gx0.at[c], sd)
            pltpu.async_copy(x1_h.at[isrc.at[c]], gx1.at[c], sd)
            pltpu.async_copy(x2_h.at[isrc.at[c]], gx2.at[c], sd)
            return 0

        lax.fori_loop(0, BLK, fire_g, 0)

        def drain_g(c, _):
            pltpu.make_async_copy(dis_h.at[pl.ds(0, 128)], dv.at[0],
                                  sd).wait()
            return 0

        lax.fori_loop(0, 4 * BLK, drain_g, 0)
        ew_mul(vv, wv, dv)
        pltpu.async_copy(vv, v_h.at[pl.ds(r, BLK)], sa)
        ew_mul(ms0, gx0, vv)
        ew_mul(ms1, gx1, vv)
        ew_mul(ms2, gx2, vv)

        def fire_s(c, _):
            pltpu.async_copy(ms0.at[c], acc0.at[idst.at[c]], sa, add=True)
            pltpu.async_copy(ms1.at[c], acc1.at[idst.at[c]], sa, add=True)
            pltpu.async_copy(ms2.at[c], acc2.at[idst.at[c]], sa, add=True)
            return 0

        lax.fori_loop(0, BLK, fire_s, 0)

        def drain_s(c, _):
            pltpu.make_async_copy(ms0.at[0], acc0.at[pl.ds(0, 128)],
                                  sa).wait()
            return 0

        lax.fori_loop(0, 3 * BLK + BLK, drain_s, 0)
        return 0

    lax.fori_loop(0, NBLK, block, 0)
    plsc.subcore_barrier()
    for acc, out in ((acc0, a0_h), (acc1, a1_h), (acc2, a2_h)):
        pltpu.sync_copy(acc.at[pl.ds(STRIPE * sid, STRIPE)], zv)
        pltpu.sync_copy(zv, out.at[pl.ds(NP * cid + STRIPE * sid, STRIPE)])


FCH = 112
CROWS = 800
SROWS = 80
NCH = SROWS


@functools.partial(
    pl.kernel,
    out_type=jax.ShapeDtypeStruct((8 * NP, 32), f32),
    mesh=_mesh,
    compiler_params=_sc_params,
    scratch_types=[
        pltpu.VMEM_SHARED((NP, 32), f32),
        pltpu.VMEM((SROWS, 32), i32),
        pltpu.VMEM((SROWS, 32), i32),
        pltpu.VMEM((SROWS, 32), f32),
        pltpu.VMEM((32, 128), f32),
        pltpu.VMEM((32, 128), f32),
        pltpu.VMEM((32, 32), f32),
        pltpu.VMEM((32, 32), f32),
        pltpu.VMEM((FCH, 32), f32),
        pltpu.SemaphoreType.DMA,
        pltpu.SemaphoreType.DMA,
        pltpu.SemaphoreType.DMA,
        pltpu.SemaphoreType.DMA,
    ],
)
def _sc_l2(src_h, dst_h, v_h, h_h, a2p_h, acc, isrc, idst, vvt,
           g0, g1, m0, m1, zf, sg0, sg1, ss0, ss1):
    cid = lax.axis_index("c")
    sid = lax.axis_index("s")
    wid = cid * 16 + sid
    base_row = wid * CROWS

    def start_g(k, gbuf, sem):
        pltpu.async_copy(h_h.at[isrc.at[k]], gbuf, sem)

    def wait_g(gbuf, sem):
        pltpu.make_async_copy(h_h.at[pl.ds(0, 32)], gbuf, sem).wait()

    def start_s(k, mbuf, sem):
        pltpu.async_copy(mbuf, acc.at[idst.at[k]], sem, add=True)

    def wait_s(mbuf, sem):
        pltpu.make_async_copy(mbuf, acc.at[pl.ds(0, 32)], sem).wait()

    def scale(k, gbuf, mbuf, p):
        def sc16(q2, _):
            so = pl.multiple_of(16 * q2, 16)
            vseg = vvt[k, pl.ds(so, 16)]
            for l in range(16):
                s16 = _splat(vseg, l)
                i = 16 * q2 + l
                for h in range(2):
                    go = pl.multiple_of(32 * p + 16 * h, 16)
                    mbuf[i, pl.ds(16 * h, 16)] = gbuf[i, pl.ds(go, 16)] * s16
            return 0

        lax.fori_loop(0, 2, sc16, 0)

    def zpass(j, _):
        zf[j >> 1, pl.ds(16 * (j & 1), 16)] = jnp.zeros((16,), f32)
        return 0

    def one_pass(p, _):
        lax.fori_loop(0, FCH * 2, zpass, 0)

        def zq(q, _):
            off = pl.multiple_of(STRIPE * sid + FCH * q, 8)
            pltpu.sync_copy(zf, acc.at[pl.ds(off, FCH)])
            return 0

        lax.fori_loop(0, 28, zq, 0)
        plsc.subcore_barrier()

        def stage(s, _):
            r = pl.multiple_of(base_row + SROWS * s, 8)
            pltpu.sync_copy(src_h.at[pl.ds(r, SROWS)], isrc)
            pltpu.sync_copy(dst_h.at[pl.ds(r, SROWS)], idst)
            pltpu.sync_copy(v_h.at[pl.ds(r, SROWS)], vvt)
            start_g(0, g0, sg0)
            start_g(1, g1, sg1)

            def pair(kk, _):
                e = 2 * kk
                o = e + 1
                wait_g(g0, sg0)

                @pl.when(kk > 0)
                def _():
                    wait_s(m0, ss0)

                scale(e, g0, m0, p)
                start_s(e, m0, ss0)

                @pl.when(e + 2 < NCH)
                def _():
                    start_g(e + 2, g0, sg0)

                wait_g(g1, sg1)

                @pl.when(kk > 0)
                def _():
                    wait_s(m1, ss1)

                scale(o, g1, m1, p)
                start_s(o, m1, ss1)

                @pl.when(o + 2 < NCH)
                def _():
                    start_g(o + 2, g1, sg1)

                return 0

            lax.fori_loop(0, NCH // 2, pair, 0)
            wait_s(m0, ss0)
            wait_s(m1, ss1)
            return 0

        lax.fori_loop(0, CROWS // SROWS, stage, 0)
        plsc.subcore_barrier()
        obase = (cid * 4 + p) * NP

        def fq(q, _):
            off = pl.multiple_of(STRIPE * sid + FCH * q, 8)
            pltpu.sync_copy(acc.at[pl.ds(off, FCH)], zf)
            oo = pl.multiple_of(obase + off, 8)
            pltpu.sync_copy(zf, a2p_h.at[pl.ds(oo, FCH)])
            return 0

        lax.fori_loop(0, 28, fq, 0)
        plsc.subcore_barrier()
        return 0

    lax.fori_loop(0, 4, one_pass, 0)


def _tc_dis_body(degp_ref, dis_ref, dis2_ref):
    x = degp_ref[0] + degp_ref[1] + 1.0
    d = lax.rsqrt(x)
    d = d * (1.5 - 0.5 * x * d * d)
    dis_ref[...] = d
    dis2_ref[...] = d * d


def _tc_dis(degp):
    degp3 = degp.reshape(2, 392, 128)
    dis, dis2 = pl.pallas_call(
        _tc_dis_body,
        grid=(7,),
        in_specs=[pl.BlockSpec((2, 56, 128), lambda i: (0, i, 0))],
        out_specs=[pl.BlockSpec((56, 128), lambda i: (i, 0)),
                   pl.BlockSpec((56, 128), lambda i: (i, 0))],
        out_shape=[jax.ShapeDtypeStruct((392, 128), f32),
                   jax.ShapeDtypeStruct((392, 128), f32)],
    )(degp3)
    return dis.reshape(NP), dis2.reshape(NP)


def _tc_h1_body(a0_ref, a1_ref, a2_ref, dis_ref, dis2_ref,
                x0_ref, x1_ref, x2_ref, w1_ref, b1_ref, o_ref):
    dis = dis_ref[...]
    dis2 = dis2_ref[...]
    h = jnp.broadcast_to(b1_ref[...], (1024, 128))
    for a_ref, x_ref, fidx in ((a0_ref, x0_ref, 0), (a1_ref, x1_ref, 1),
                               (a2_ref, x2_ref, 2)):
        agg = dis * (a_ref[0] + a_ref[1]) + dis2 * x_ref[...]
        h = h + agg * w1_ref[fidx:fidx + 1, :]
    o_ref[...] = h * _sigmoid(h)


def _tc_h1(a0, a1, a2, dis, dis2, x0, x1, x2, w1p, b1p):
    col = pl.BlockSpec((1024, 1), lambda i: (i, 0))
    par = pl.BlockSpec((2, 1024, 1), lambda i: (0, i, 0))
    return pl.pallas_call(
        _tc_h1_body,
        grid=(49,),
        in_specs=[par, par, par, col, col, col, col, col,
                  pl.BlockSpec((3, 128), lambda i: (0, 0)),
                  pl.BlockSpec((1, 128), lambda i: (0, 0))],
        out_specs=pl.BlockSpec((1024, 128), lambda i: (i, 0)),
        out_shape=jax.ShapeDtypeStruct((NP, 128), f32),
    )(a0.reshape(2, NP, 1), a1.reshape(2, NP, 1), a2.reshape(2, NP, 1),
      dis.reshape(NP, 1), dis2.reshape(NP, 1),
      x0.reshape(NP, 1), x1.reshape(NP, 1), x2.reshape(NP, 1), w1p, b1p)


def _tc_final_body(a2p_ref, h_ref,
                   dis_ref, dis2_ref, batch_ref, w2_ref, b2_ref,
                   lw1_ref, lb1_ref, lw2_ref, lb2_ref,
                   out_ref, sums, cnt):
    pid = pl.program_id(0)

    @pl.when(pid == 0)
    def _():
        sums[...] = jnp.zeros((16, 256), f32)
        cnt[...] = jnp.zeros((16, 128), f32)

    dis = dis_ref[...]
    dis2 = dis2_ref[...]
    acc = jnp.broadcast_to(b2_ref[...], (1024, 256))
    for p in range(4):
        aggp = (dis * (a2p_ref[0, p] + a2p_ref[1, p])
                + dis2 * h_ref[:, 32 * p:32 * p + 32])
        acc = acc + jnp.dot(aggp, w2_ref[32 * p:32 * p + 32, :],
                            precision=_HI, preferred_element_type=f32)
    h2 = acc * _sigmoid(acc)
    bb = batch_ref[...]
    io = lax.broadcasted_iota(i32, (1024, 16), 1)
    oh = jnp.where(bb == io, 1.0, 0.0).astype(f32)
    dn = (((0,), (0,)), ((), ()))
    sums[...] += lax.dot_general(oh, h2, dn, precision=_HI,
                                 preferred_element_type=f32)
    ones = jnp.ones((1024, 128), f32)
    cnt[...] += lax.dot_general(oh, ones, dn, precision=_HI,
                                preferred_element_type=f32)

    @pl.when(pid == 48)
    def _():
        pooled = sums[...] / jnp.maximum(cnt[...][:, 0:1], 1.0)
        ph = jnp.dot(pooled, lw1_ref[...], precision=_HI,
                     preferred_element_type=f32)
        ph = ph + lb1_ref[...]
        ph = ph * _sigmoid(ph)
        res = jnp.dot(ph, lw2_ref[...], precision=_HI,
                      preferred_element_type=f32)
        out_ref[...] = res + lb2_ref[...]


def _tc_final(a2p, h1p, dis, dis2, batchp, w2p, b2p, lw1p, lb1p, lw2p, lb2p):
    col = pl.BlockSpec((1024, 1), lambda i: (i, 0))
    whole = lambda *shape: pl.BlockSpec(shape, lambda i: tuple(0 for _ in shape))
    return pl.pallas_call(
        _tc_final_body,
        grid=(49,),
        in_specs=[pl.BlockSpec((2, 4, 1024, 32), lambda i: (0, 0, i, 0)),
                  pl.BlockSpec((1024, 128), lambda i: (i, 0)), col, col, col,
                  whole(128, 256), whole(1, 256),
                  whole(256, 128), whole(1, 128),
                  whole(128, 128), whole(1, 128)],
        out_specs=pl.BlockSpec((16, 128), lambda i: (0, 0)),
        out_shape=jax.ShapeDtypeStruct((16, 128), f32),
        scratch_shapes=[pltpu.VMEM((16, 256), f32),
                        pltpu.VMEM((16, 128), f32)],
    )(a2p, h1p, dis.reshape(NP, 1), dis2.reshape(NP, 1), batchp,
      w2p, b2p, lw1p, lb1p, lw2p, lb2p)


def kernel(x, edge_index, edge_attr, batch, W1, b1, W2, b2,
           LW1, Lb1, LW2, Lb2):
    src = edge_index[0].astype(i32)
    dst = edge_index[1].astype(i32)
    w = edge_attr.astype(f32)
    pad = EP - E
    src2 = jnp.concatenate([src, jnp.zeros((pad,), i32)]).reshape(ER, 128)
    dst2 = jnp.concatenate([dst, jnp.zeros((pad,), i32)]).reshape(ER, 128)
    w2e = jnp.concatenate([w, jnp.zeros((pad,), f32)]).reshape(ER, 128)

    degp = _sc_deg(dst2, w2e)
    dis, dis2 = _tc_dis(degp)

    xp = jnp.pad(x, ((0, NP - N), (0, 0)))
    x0, x1, x2 = xp[:, 0], xp[:, 1], xp[:, 2]
    v2, a0, a1, a2 = _sc_l1(src2, dst2, w2e, dis, x0, x1, x2)

    w1p = jnp.pad(W1, ((0, 0), (0, 28)))
    b1p = jnp.pad(b1, (0, 28)).reshape(1, 128)
    h1p = _tc_h1(a0, a1, a2, dis, dis2, x0, x1, x2, w1p, b1p)

    a2p = _sc_l2(src2.reshape(EP // 32, 32), dst2.reshape(EP // 32, 32),
                 v2.reshape(EP // 32, 32), h1p).reshape(2, 4, NP, 32)

    batchp = jnp.concatenate(
        [batch.astype(i32), jnp.full((NP - N,), G, i32)]).reshape(NP, 1)
    w2p = jnp.pad(W2, ((0, 28), (0, 56)))
    b2p = jnp.pad(b2, (0, 56)).reshape(1, 256)
    lw1p = jnp.pad(LW1, ((0, 56), (0, 28)))
    lb1p = jnp.pad(Lb1, (0, 28)).reshape(1, 128)
    lw2p = jnp.pad(LW2, ((0, 28), (0, 127)))
    lb2p = jnp.pad(Lb2, (0, 127)).reshape(1, 128)

    outf = _tc_final(a2p, h1p, dis, dis2, batchp,
                     w2p, b2p, lw1p, lb1p, lw2p, lb2p)
    return outf[:, 0]


# speedup vs baseline: 5.4860x; 1.0017x over previous
import functools

import jax
import jax.numpy as jnp
from jax import lax
from jax.experimental import pallas as pl
from jax.experimental.pallas import tpu as pltpu
from jax.experimental.pallas import tpu_sc as plsc

N = 50000
E = 800000
G = 16

NP = 50176
EP = 819200
ER = EP // 128
TILES = 32
ROWS_PER_TILE = ER // TILES
BLK = 8
NBLK = ROWS_PER_TILE // BLK
STRIPE = NP // 16

_mesh = plsc.VectorSubcoreMesh(core_axis_name="c", subcore_axis_name="s")
_sc_params = pltpu.CompilerParams(use_tc_tiling_on_sc=False)
f32 = jnp.float32
i32 = jnp.int32
_HI = lax.Precision.HIGHEST


def _sigmoid(x):
    return 0.5 + 0.5 * jnp.tanh(0.5 * x)


def _zero_vec(ref, nwords):
    def body(j, _):
        ref[pl.ds(16 * j, 16)] = jnp.zeros((16,), f32)
        return 0
    lax.fori_loop(0, nwords // 16, body, 0)


_GDN = lax.GatherDimensionNumbers(
    offset_dims=(), collapsed_slice_dims=(0,), start_index_map=(0,))


def _splat(vec16, lane):
    idx = jnp.full((16, 1), lane, i32)
    return lax.gather(vec16, idx, _GDN, slice_sizes=(1,),
                      mode=lax.GatherScatterMode.PROMISE_IN_BOUNDS)


@functools.partial(
    pl.kernel,
    out_type=jax.ShapeDtypeStruct((2 * NP,), f32),
    mesh=_mesh,
    compiler_params=_sc_params,
    scratch_types=[
        pltpu.VMEM_SHARED((NP,), f32),
        pltpu.VMEM((BLK, 128), i32),
        pltpu.VMEM((BLK, 128), f32),
        pltpu.VMEM((STRIPE,), f32),
        pltpu.SemaphoreType.DMA,
    ],
)
def _sc_deg(dst_h, w_h, degp_h, acc, idst, wv, zv, sa):
    cid = lax.axis_index("c")
    sid = lax.axis_index("s")
    _zero_vec(zv, STRIPE)
    pltpu.sync_copy(zv, acc.at[pl.ds(STRIPE * sid, STRIPE)])
    plsc.subcore_barrier()
    row0 = (cid * 16 + sid) * ROWS_PER_TILE

    def block(b, _):
        r = row0 + b * BLK
        pltpu.sync_copy(dst_h.at[pl.ds(r, BLK)], idst)
        pltpu.sync_copy(w_h.at[pl.ds(r, BLK)], wv)

        def chunk(c, _):
            pltpu.async_copy(wv.at[c], acc.at[idst.at[c]], sa, add=True)
            return 0

        lax.fori_loop(0, BLK, chunk, 0)

        def drain(c, _):
            pltpu.make_async_copy(wv.at[0], acc.at[pl.ds(0, 128)], sa).wait()
            return 0

        lax.fori_loop(0, BLK, drain, 0)
        return 0

    lax.fori_loop(0, NBLK, block, 0)
    plsc.subcore_barrier()
    pltpu.sync_copy(acc.at[pl.ds(STRIPE * sid, STRIPE)], zv)
    pltpu.sync_copy(zv, degp_h.at[pl.ds(NP * cid + STRIPE * sid, STRIPE)])


@functools.partial(
    pl.kernel,
    out_type=(
        jax.ShapeDtypeStruct((ER, 128), f32),
        jax.ShapeDtypeStruct((2 * NP,), f32),
        jax.ShapeDtypeStruct((2 * NP,), f32),
        jax.ShapeDtypeStruct((2 * NP,), f32),
    ),
    mesh=_mesh,
    compiler_params=_sc_params,
    scratch_types=[
        pltpu.VMEM_SHARED((NP,), f32),
        pltpu.VMEM_SHARED((NP,), f32),
        pltpu.VMEM_SHARED((NP,), f32),
        pltpu.VMEM((BLK, 128), i32),
        pltpu.VMEM((BLK, 128), i32),
        pltpu.VMEM((BLK, 128), f32),
        pltpu.VMEM((BLK, 128), f32),
        pltpu.VMEM((BLK, 128), f32),
        pltpu.VMEM((BLK, 128), f32),
        pltpu.VMEM((BLK, 128), f32),
        pltpu.VMEM((BLK, 128), f32),
        pltpu.VMEM((BLK, 128), f32),
        pltpu.VMEM((BLK, 128), f32),
        pltpu.VMEM((BLK, 128), f32),
        pltpu.VMEM((STRIPE,), f32),
        pltpu.SemaphoreType.DMA,
        pltpu.SemaphoreType.DMA,
    ],
)
def _sc_l1(src_h, dst_h, w_h, dis_h, x0_h, x1_h, x2_h,
           v_h, a0_h, a1_h, a2_h,
           acc0, acc1, acc2, isrc, idst, wv, dv, vv,
           gx0, gx1, gx2, ms0, ms1, ms2, zv, sd, sa):
    cid = lax.axis_index("c")
    sid = lax.axis_index("s")
    _zero_vec(zv, STRIPE)
    for acc in (acc0, acc1, acc2):
        pltpu.sync_copy(zv, acc.at[pl.ds(STRIPE * sid, STRIPE)])
    plsc.subcore_barrier()
    row0 = (cid * 16 + sid) * ROWS_PER_TILE

    def ew_mul(dst_ref, a_ref, b_ref):
        def body(j, _):
            rr = j >> 3
            sl = pl.ds(16 * (j & 7), 16)
            dst_ref[rr, sl] = a_ref[rr, sl] * b_ref[rr, sl]
            return 0
        lax.fori_loop(0, BLK * 8, body, 0)

    def block(b, _):
        r = row0 + b * BLK
        pltpu.sync_copy(src_h.at[pl.ds(r, BLK)], isrc)
        pltpu.sync_copy(dst_h.at[pl.ds(r, BLK)], idst)
        pltpu.sync_copy(w_h.at[pl.ds(r, BLK)], wv)

        def fire_g(c, _):
            pltpu.async_copy(dis_h.at[isrc.at[c]], dv.at[c], sd)
            pltpu.async_copy(x0_h.at[isrc.at[c]], gx0.at[c], sd)
            pltpu.async_copy(x1_h.at[isrc.at[c]], gx1.at[c], sd)
            pltpu.async_copy(x2_h.at[isrc.at[c]], gx2.at[c], sd)
            return 0

        lax.fori_loop(0, BLK, fire_g, 0)

        def drain_g(c, _):
            pltpu.make_async_copy(dis_h.at[pl.ds(0, 128)], dv.at[0],
                                  sd).wait()
            return 0

        lax.fori_loop(0, 4 * BLK, drain_g, 0)
        ew_mul(vv, wv, dv)
        pltpu.async_copy(vv, v_h.at[pl.ds(r, BLK)], sa)
        ew_mul(ms0, gx0, vv)
        ew_mul(ms1, gx1, vv)
        ew_mul(ms2, gx2, vv)

        def fire_s(c, _):
            pltpu.async_copy(ms0.at[c], acc0.at[idst.at[c]], sa, add=True)
            pltpu.async_copy(ms1.at[c], acc1.at[idst.at[c]], sa, add=True)
            pltpu.async_copy(ms2.at[c], acc2.at[idst.at[c]], sa, add=True)
            return 0

        lax.fori_loop(0, BLK, fire_s, 0)

        def drain_s(c, _):
            pltpu.make_async_copy(ms0.at[0], acc0.at[pl.ds(0, 128)],
                                  sa).wait()
            return 0

        lax.fori_loop(0, 3 * BLK + BLK, drain_s, 0)
        return 0

    lax.fori_loop(0, NBLK, block, 0)
    plsc.subcore_barrier()
    for acc, out in ((acc0, a0_h), (acc1, a1_h), (acc2, a2_h)):
        pltpu.sync_copy(acc.at[pl.ds(STRIPE * sid, STRIPE)], zv)
        pltpu.sync_copy(zv, out.at[pl.ds(NP * cid + STRIPE * sid, STRIPE)])


FCH = 112
CROWS = 800
SROWS = 80
NCH = SROWS


@functools.partial(
    pl.kernel,
    out_type=jax.ShapeDtypeStruct((8 * NP, 32), f32),
    mesh=_mesh,
    compiler_params=_sc_params,
    scratch_types=[
        pltpu.VMEM_SHARED((NP, 32), f32),
        pltpu.VMEM((SROWS, 32), i32),
        pltpu.VMEM((SROWS, 32), i32),
        pltpu.VMEM((SROWS, 32), f32),
        pltpu.VMEM((32, 128), f32),
        pltpu.VMEM((32, 128), f32),
        pltpu.VMEM((32, 32), f32),
        pltpu.VMEM((32, 32), f32),
        pltpu.VMEM((FCH, 32), f32),
        pltpu.SemaphoreType.DMA,
        pltpu.SemaphoreType.DMA,
        pltpu.SemaphoreType.DMA,
        pltpu.SemaphoreType.DMA,
    ],
)
def _sc_l2(src_h, dst_h, v_h, h_h, a2p_h, acc, isrc, idst, vvt,
           g0, g1, m0, m1, zf, sg0, sg1, ss0, ss1):
    cid = lax.axis_index("c")
    sid = lax.axis_index("s")
    wid = cid * 16 + sid
    base_row = wid * CROWS

    def start_g(k, gbuf, sem):
        pltpu.async_copy(h_h.at[isrc.at[k]], gbuf, sem)

    def wait_g(gbuf, sem):
        pltpu.make_async_copy(h_h.at[pl.ds(0, 32)], gbuf, sem).wait()

    def start_s(k, mbuf, sem):
        pltpu.async_copy(mbuf, acc.at[idst.at[k]], sem, add=True)

    def wait_s(mbuf, sem):
        pltpu.make_async_copy(mbuf, acc.at[pl.ds(0, 32)], sem).wait()

    def scale(k, gbuf, mbuf, p):
        def sc16(q2, _):
            so = pl.multiple_of(16 * q2, 16)
            vseg = vvt[k, pl.ds(so, 16)]
            for l in range(16):
                s16 = _splat(vseg, l)
                i = 16 * q2 + l
                for h in range(2):
                    go = pl.multiple_of(32 * p + 16 * h, 16)
                    mbuf[i, pl.ds(16 * h, 16)] = gbuf[i, pl.ds(go, 16)] * s16
            return 0

        lax.fori_loop(0, 2, sc16, 0)

    def zpass(j, _):
        zf[j >> 1, pl.ds(16 * (j & 1), 16)] = jnp.zeros((16,), f32)
        return 0

    def one_pass(p, _):
        lax.fori_loop(0, FCH * 2, zpass, 0)

        def zq(q, _):
            off = pl.multiple_of(STRIPE * sid + FCH * q, 8)
            pltpu.sync_copy(zf, acc.at[pl.ds(off, FCH)])
            return 0

        lax.fori_loop(0, 28, zq, 0)
        plsc.subcore_barrier()

        def stage(s, _):
            r = pl.multiple_of(base_row + SROWS * s, 8)
            pltpu.sync_copy(src_h.at[pl.ds(r, SROWS)], isrc)
            pltpu.sync_copy(dst_h.at[pl.ds(r, SROWS)], idst)
            pltpu.sync_copy(v_h.at[pl.ds(r, SROWS)], vvt)
            start_g(0, g0, sg0)
            start_g(1, g1, sg1)

            def pair(kk, _):
                e = 2 * kk
                o = e + 1
                wait_g(g0, sg0)

                @pl.when(kk > 0)
                def _():
                    wait_s(m0, ss0)

                scale(e, g0, m0, p)
                start_s(e, m0, ss0)

                @pl.when(e + 2 < NCH)
                def _():
                    start_g(e + 2, g0, sg0)

                wait_g(g1, sg1)

                @pl.when(kk > 0)
                def _():
                    wait_s(m1, ss1)

                scale(o, g1, m1, p)
                start_s(o, m1, ss1)

                @pl.when(o + 2 < NCH)
                def _():
                    start_g(o + 2, g1, sg1)

                return 0

            lax.fori_loop(0, NCH // 2, pair, 0)
            wait_s(m0, ss0)
            wait_s(m1, ss1)
            return 0

        lax.fori_loop(0, CROWS // SROWS, stage, 0)
        plsc.subcore_barrier()
        obase = (cid * 4 + p) * NP

        def fq(q, _):
            off = pl.multiple_of(STRIPE * sid + FCH * q, 8)
            pltpu.sync_copy(acc.at[pl.ds(off, FCH)], zf)
            oo = pl.multiple_of(obase + off, 8)
            pltpu.sync_copy(zf, a2p_h.at[pl.ds(oo, FCH)])
            return 0

        lax.fori_loop(0, 28, fq, 0)
        plsc.subcore_barrier()
        return 0

    lax.fori_loop(0, 4, one_pass, 0)


def _tc_dis_body(degp_ref, dis_ref, dis2_ref):
    x = degp_ref[0] + degp_ref[1] + 1.0
    d = lax.rsqrt(x)
    d = d * (1.5 - 0.5 * x * d * d)
    dis_ref[...] = d
    dis2_ref[...] = d * d


def _tc_dis(degp):
    degp3 = degp.reshape(2, 392, 128)
    dis, dis2 = pl.pallas_call(
        _tc_dis_body,
        grid=(7,),
        in_specs=[pl.BlockSpec((2, 56, 128), lambda i: (0, i, 0))],
        out_specs=[pl.BlockSpec((56, 128), lambda i: (i, 0)),
                   pl.BlockSpec((56, 128), lambda i: (i, 0))],
        out_shape=[jax.ShapeDtypeStruct((392, 128), f32),
                   jax.ShapeDtypeStruct((392, 128), f32)],
    )(degp3)
    return dis.reshape(NP), dis2.reshape(NP)


def _tc_h1_body(a0_ref, a1_ref, a2_ref, dis_ref, dis2_ref,
                x0_ref, x1_ref, x2_ref, w1_ref, b1_ref, o_ref):
    dis = dis_ref[...]
    dis2 = dis2_ref[...]
    h = jnp.broadcast_to(b1_ref[...], (1024, 128))
    for a_ref, x_ref, fidx in ((a0_ref, x0_ref, 0), (a1_ref, x1_ref, 1),
                               (a2_ref, x2_ref, 2)):
        agg = dis * (a_ref[0] + a_ref[1]) + dis2 * x_ref[...]
        h = h + agg * w1_ref[fidx:fidx + 1, :]
    o_ref[...] = h * _sigmoid(h)


def _tc_h1(a0, a1, a2, dis, dis2, x0, x1, x2, w1p, b1p):
    col = pl.BlockSpec((1024, 1), lambda i: (i, 0))
    par = pl.BlockSpec((2, 1024, 1), lambda i: (0, i, 0))
    return pl.pallas_call(
        _tc_h1_body,
        grid=(49,),
        in_specs=[par, par, par, col, col, col, col, col,
                  pl.BlockSpec((3, 128), lambda i: (0, 0)),
                  pl.BlockSpec((1, 128), lambda i: (0, 0))],
        out_specs=pl.BlockSpec((1024, 128), lambda i: (i, 0)),
        out_shape=jax.ShapeDtypeStruct((NP, 128), f32),
    )(a0.reshape(2, NP, 1), a1.reshape(2, NP, 1), a2.reshape(2, NP, 1),
      dis.reshape(NP, 1), dis2.reshape(NP, 1),
      x0.reshape(NP, 1), x1.reshape(NP, 1), x2.reshape(NP, 1), w1p, b1p)


def _tc_final_body(a2p_ref, h_ref,
                   dis_ref, dis2_ref, batch_ref, w2_ref, b2_ref,
                   lw1_ref, lb1_ref, lw2_ref, lb2_ref,
                   out_ref, sums, comp, cnt):
    pid = pl.program_id(0)

    @pl.when(pid == 0)
    def _():
        sums[...] = jnp.zeros((16, 256), f32)
        comp[...] = jnp.zeros((16, 256), f32)
        cnt[...] = jnp.zeros((16, 128), f32)

    dis = dis_ref[...]
    dis2 = dis2_ref[...]
    acc = jnp.broadcast_to(b2_ref[...], (1024, 256))
    for p in range(4):
        aggp = (dis * (a2p_ref[0, p] + a2p_ref[1, p])
                + dis2 * h_ref[:, 32 * p:32 * p + 32])
        acc = acc + jnp.dot(aggp, w2_ref[32 * p:32 * p + 32, :],
                            precision=_HI, preferred_element_type=f32)
    h2 = acc * _sigmoid(acc)
    bb = batch_ref[...]
    io = lax.broadcasted_iota(i32, (1024, 16), 1)
    oh = jnp.where(bb == io, 1.0, 0.0).astype(f32)
    dn = (((0,), (0,)), ((), ()))
    blk = lax.dot_general(oh, h2, dn, precision=_HI,
                          preferred_element_type=f32)
    y = blk - comp[...]
    s0 = sums[...]
    t = s0 + y
    comp[...] = (t - s0) - y
    sums[...] = t
    ones = jnp.ones((1024, 128), f32)
    cnt[...] += lax.dot_general(oh, ones, dn, precision=_HI,
                                preferred_element_type=f32)

    @pl.when(pid == 48)
    def _():
        pooled = sums[...] / jnp.maximum(cnt[...][:, 0:1], 1.0)
        ph = jnp.dot(pooled, lw1_ref[...], precision=_HI,
                     preferred_element_type=f32)
        ph = ph + lb1_ref[...]
        ph = ph * _sigmoid(ph)
        res = jnp.dot(ph, lw2_ref[...], precision=_HI,
                      preferred_element_type=f32)
        out_ref[...] = res + lb2_ref[...]


def _tc_final(a2p, h1p, dis, dis2, batchp, w2p, b2p, lw1p, lb1p, lw2p, lb2p):
    col = pl.BlockSpec((1024, 1), lambda i: (i, 0))
    whole = lambda *shape: pl.BlockSpec(shape, lambda i: tuple(0 for _ in shape))
    return pl.pallas_call(
        _tc_final_body,
        grid=(49,),
        in_specs=[pl.BlockSpec((2, 4, 1024, 32), lambda i: (0, 0, i, 0)),
                  pl.BlockSpec((1024, 128), lambda i: (i, 0)), col, col, col,
                  whole(128, 256), whole(1, 256),
                  whole(256, 128), whole(1, 128),
                  whole(128, 128), whole(1, 128)],
        out_specs=pl.BlockSpec((16, 128), lambda i: (0, 0)),
        out_shape=jax.ShapeDtypeStruct((16, 128), f32),
        scratch_shapes=[pltpu.VMEM((16, 256), f32),
                        pltpu.VMEM((16, 256), f32),
                        pltpu.VMEM((16, 128), f32)],
    )(a2p, h1p, dis.reshape(NP, 1), dis2.reshape(NP, 1), batchp,
      w2p, b2p, lw1p, lb1p, lw2p, lb2p)


def kernel(x, edge_index, edge_attr, batch, W1, b1, W2, b2,
           LW1, Lb1, LW2, Lb2):
    src = edge_index[0].astype(i32)
    dst = edge_index[1].astype(i32)
    w = edge_attr.astype(f32)
    pad = EP - E
    src2 = jnp.concatenate([src, jnp.zeros((pad,), i32)]).reshape(ER, 128)
    dst2 = jnp.concatenate([dst, jnp.zeros((pad,), i32)]).reshape(ER, 128)
    w2e = jnp.concatenate([w, jnp.zeros((pad,), f32)]).reshape(ER, 128)

    degp = _sc_deg(dst2, w2e)
    dis, dis2 = _tc_dis(degp)

    xp = jnp.pad(x, ((0, NP - N), (0, 0)))
    x0, x1, x2 = xp[:, 0], xp[:, 1], xp[:, 2]
    v2, a0, a1, a2 = _sc_l1(src2, dst2, w2e, dis, x0, x1, x2)

    w1p = jnp.pad(W1, ((0, 0), (0, 28)))
    b1p = jnp.pad(b1, (0, 28)).reshape(1, 128)
    h1p = _tc_h1(a0, a1, a2, dis, dis2, x0, x1, x2, w1p, b1p)

    a2p = _sc_l2(src2.reshape(EP // 32, 32), dst2.reshape(EP // 32, 32),
                 v2.reshape(EP // 32, 32), h1p).reshape(2, 4, NP, 32)

    batchp = jnp.concatenate(
        [batch.astype(i32), jnp.full((NP - N,), G, i32)]).reshape(NP, 1)
    w2p = jnp.pad(W2, ((0, 28), (0, 56)))
    b2p = jnp.pad(b2, (0, 56)).reshape(1, 256)
    lw1p = jnp.pad(LW1, ((0, 56), (0, 28)))
    lb1p = jnp.pad(Lb1, (0, 28)).reshape(1, 128)
    lw2p = jnp.pad(LW2, ((0, 28), (0, 127)))
    lb2p = jnp.pad(Lb2, (0, 127)).reshape(1, 128)

    outf = _tc_final(a2p, h1p, dis, dis2, batchp,
                     w2p, b2p, lw1p, lb1p, lw2p, lb2p)
    return outf[:, 0]
